# baseline (speedup 1.0000x reference)
import functools

import jax
import jax.numpy as jnp
from jax import lax
from jax.experimental import pallas as pl
from jax.experimental.pallas import tpu as pltpu
from jax.experimental.pallas import tpu_sc as plsc

NUM_ENT = 50000
NUM_REL = 474
D = 128
NODE_D = 32
B = 128
NB = 10
THRESH = 1373

VOCAB_CHUNK = 2048


def _dense_body(h_ref, e_ref, nbr_ref, r_ref, nod_ref, wve_ref, wvn_ref,
                bv_ref, f1a_ref, f1b_ref, b1_ref, f2_ref, b2_ref, ent_ref,
                yc_ref, out_s):
    @pl.when(pl.program_id(0) == 0)
    def _():
        nbr = nbr_ref[...]
        valid = jax.lax.broadcasted_iota(jnp.int32, (B, 16), 1) < NB
        cnt = jnp.sum(jnp.where(valid & (nbr >= THRESH), 1.0, 0.0),
                      axis=1, keepdims=True)
        node = (NB - cnt) * nod_ref[0:1, :] + cnt * nod_ref[1:2, :]
        V = (h_ref[...]
             + jnp.dot(e_ref[...], wve_ref[...], preferred_element_type=jnp.float32)
             + jnp.dot(node, wvn_ref[...], preferred_element_type=jnp.float32)
             + NB * bv_ref[...])
        z1 = jnp.maximum(
            jnp.dot(V, f1a_ref[...], preferred_element_type=jnp.float32)
            + jnp.dot(r_ref[...], f1b_ref[...], preferred_element_type=jnp.float32)
            + b1_ref[...], 0.0)
        out_s[...] = (jnp.dot(z1, f2_ref[...], preferred_element_type=jnp.float32)
                      + b2_ref[...])

    logits = jax.lax.dot_general(out_s[...], ent_ref[...],
                                 (((1,), (1,)), ((), ())),
                                 preferred_element_type=jnp.float32)
    yc_ref[...] = jax.nn.sigmoid(logits)


def _dense_stage(h_emb, e_sum, nbr_ids, r_emb, nod_embed, WV, bV,
                 fc1_w, fc1_b, fc2_w, fc2_b, ent_embed):
    n_chunks = pl.cdiv(NUM_ENT, VOCAB_CHUNK)
    const = lambda shape: pl.BlockSpec(shape, lambda i: (0, 0))
    return pl.pallas_call(
        _dense_body,
        grid=(n_chunks,),
        in_specs=[
            const((B, D)),
            const((B, D)),
            const((B, 16)),
            const((B, D)),
            const((2, NODE_D)),
            const((D, D)),
            const((NODE_D, D)),
            const((1, D)),
            const((D, D)),
            const((D, D)),
            const((1, D)),
            const((D, D)),
            const((1, D)),
            pl.BlockSpec((VOCAB_CHUNK, D), lambda i: (i, 0)),
        ],
        out_specs=pl.BlockSpec((B, VOCAB_CHUNK), lambda i: (0, i)),
        out_shape=jax.ShapeDtypeStruct((B, NUM_ENT), jnp.float32),
        scratch_shapes=[pltpu.VMEM((B, D), jnp.float32)],
    )(h_emb, e_sum, nbr_ids, r_emb, nod_embed,
      WV[:, :D].T, WV[:, D:].T, bV.reshape(1, D),
      fc1_w[:, :D].T, fc1_w[:, D:].T, fc1_b.reshape(1, D),
      fc2_w.T, fc2_b.reshape(1, D), ent_embed)


_NW_ACT = 8
_G = B // _NW_ACT


def _gather_stage(src, rel, t_idxs, ent_embed, rel_embed):
    mesh = plsc.VectorSubcoreMesh(core_axis_name="c", subcore_axis_name="s",
                                  num_cores=2, num_subcores=16)

    @functools.partial(
        pl.kernel,
        out_type=[
            jax.ShapeDtypeStruct((B, D), jnp.float32),
            jax.ShapeDtypeStruct((B, D), jnp.float32),
            jax.ShapeDtypeStruct((B, D), jnp.float32),
            jax.ShapeDtypeStruct((B, 16), jnp.int32),
        ],
        mesh=mesh,
        compiler_params=pltpu.CompilerParams(use_tc_tiling_on_sc=False),
        scratch_types=[
            pltpu.VMEM((_G,), jnp.int32),
            pltpu.VMEM((_G,), jnp.int32),
            pltpu.VMEM((_G, 16), jnp.int32),
            pltpu.VMEM((_G, D), jnp.float32),
            pltpu.VMEM((_G, D), jnp.float32),
            pltpu.VMEM((_G, 16, D), jnp.float32),
            pltpu.VMEM((_G, D), jnp.float32),
            [pltpu.VMEM((16,), jnp.int32) for _ in range(_G)],
            pltpu.SemaphoreType.DMA,
            pltpu.SemaphoreType.DMA,
        ],
    )
    def k(src_h, rel_h, t16_h, ent_h, relemb_h,
          h_out, esum_out, r_out, nbr_out,
          src_v, rel_v, nbr_v, h_v, r_v, g_v, es_v, idx_vs, sem, sem2):
        wid = lax.axis_index("s") * 2 + lax.axis_index("c")

        @pl.when(wid < _NW_ACT)
        def _():
            base = wid * _G
            pltpu.sync_copy(src_h.at[pl.ds(base, _G)], src_v)
            pltpu.sync_copy(rel_h.at[pl.ds(base, _G)], rel_v)
            cp_h = pltpu.async_copy(ent_h.at[src_v], h_v, sem2)
            cp_r = pltpu.async_copy(relemb_h.at[rel_v], r_v, sem2)
            cp_n = pltpu.async_copy(t16_h.at[src_v], nbr_v, sem)
            cp_n.wait()
            cps = []
            for i in range(_G):
                idx_vs[i][...] = nbr_v[i]
                cps.append(
                    pltpu.async_copy(ent_h.at[idx_vs[i]], g_v.at[i], sem))
            cp_h.wait()
            cp_r.wait()
            for c in cps:
                c.wait()
            for i in range(_G):
                for c8 in range(D // 16):
                    sl = pl.ds(c8 * 16, 16)
                    acc = g_v[i, 0, sl]
                    for j in range(1, NB):
                        acc = acc + g_v[i, j, sl]
                    es_v[i, sl] = acc
            pltpu.sync_copy(h_v, h_out.at[pl.ds(base, _G)])
            pltpu.sync_copy(es_v, esum_out.at[pl.ds(base, _G)])
            pltpu.sync_copy(r_v, r_out.at[pl.ds(base, _G)])
            pltpu.sync_copy(nbr_v, nbr_out.at[pl.ds(base, _G)])

    t16 = jnp.pad(t_idxs, ((0, 0), (0, 16 - NB)))
    return k(src, rel, t16, ent_embed, rel_embed)


def kernel(src, rel, t_idxs, ent_embed, rel_embed, nod_embed,
           WQ, bQ, WK, bK, WV, bV, fc1_w, fc1_b, fc2_w, fc2_b):
    h_emb, e_sum, r_emb, nbr_ids = _gather_stage(src, rel, t_idxs,
                                                 ent_embed, rel_embed)
    return _dense_stage(h_emb, e_sum, nbr_ids, r_emb, nod_embed,
                        WV, bV, fc1_w, fc1_b, fc2_w, fc2_b, ent_embed)
